# baseline (speedup 1.0000x reference)
import functools

import jax
import jax.numpy as jnp
from jax.experimental import pallas as pl
from jax.experimental.pallas import tpu as pltpu

_CAPF = 1.25
_K = 8
_BT = 256


def _router_kernel(x_ref, w_ref,
                   idx_ref, wts_ref, mask_ref,
                   counts_ref, avgp_ref, ent_ref, gini_ref, ctr_ref, drop_ref,
                   vbuf_ref, acc_counts, acc_probs, acc_ent, acc_keep, acc_drop,
                   *, bt, e, k, nt, cap):
    i = pl.program_id(0)
    nsteps = pl.num_programs(0)

    @pl.when(i == 0)
    def _init():
        acc_counts[...] = jnp.zeros_like(acc_counts)
        acc_probs[...] = jnp.zeros_like(acc_probs)
        acc_ent[...] = jnp.zeros_like(acc_ent)
        acc_keep[...] = jnp.zeros_like(acc_keep)
        acc_drop[...] = jnp.zeros_like(acc_drop)

    xb = x_ref[...]
    logits = jax.lax.dot_general(xb, w_ref[...], (((1,), (1,)), ((), ())),
                                 preferred_element_type=jnp.float32)
    m = jnp.max(logits, axis=-1, keepdims=True)
    el = jnp.exp(logits - m)
    probs = el / jnp.sum(el, axis=-1, keepdims=True)

    lane = jax.lax.broadcasted_iota(jnp.int32, (bt, e), 1)
    cur = probs
    selected = jnp.zeros((bt, e), jnp.bool_)
    for kk in range(k):
        mk = jnp.max(cur, axis=-1, keepdims=True)
        ik = jnp.min(jnp.where(cur == mk, lane, e), axis=-1, keepdims=True)
        oh = lane == ik
        idx_ref[:, kk:kk + 1] = ik
        vbuf_ref[:, kk:kk + 1] = mk
        selected = selected | oh
        cur = jnp.where(oh, -jnp.inf, cur)

    h = selected.astype(jnp.float32)
    sv = jnp.sum(jnp.where(selected, probs, 0.0), axis=1, keepdims=True)

    r2 = jax.lax.broadcasted_iota(jnp.int32, (bt, bt), 0)
    c2 = jax.lax.broadcasted_iota(jnp.int32, (bt, bt), 1)
    tri = (c2 < r2).astype(jnp.float32)
    excl = jax.lax.dot_general(tri, h, (((1,), (0,)), ((), ())),
                               preferred_element_type=jnp.float32)
    excl = excl + acc_counts[...]

    keepe = jnp.where(selected & (excl < cap), 1.0, 0.0)
    masksum = jnp.sum(keepe, axis=1, keepdims=True)
    wscale = (1.0 / sv) / (masksum + 1e-10)
    for kk in range(k):
        oh = lane == idx_ref[:, kk:kk + 1]
        mask_k = jnp.sum(jnp.where(oh, keepe, 0.0), axis=1, keepdims=True)
        mask_ref[:, kk:kk + 1] = mask_k
        wts_ref[:, kk:kk + 1] = vbuf_ref[:, kk:kk + 1] * wscale * mask_k

    acc_counts[...] = acc_counts[...] + jnp.sum(h, axis=0, keepdims=True)
    acc_probs[...] = acc_probs[...] + jnp.sum(probs, axis=0, keepdims=True)
    acc_ent[...] = acc_ent[...] + jnp.sum(-probs * jnp.log(probs + 1e-10),
                                          keepdims=True)
    acc_keep[...] = acc_keep[...] + jnp.sum(keepe, axis=0, keepdims=True)
    acc_drop[...] = acc_drop[...] + (
        float(bt * k) - jnp.sum(masksum, keepdims=True))

    @pl.when(i == nsteps - 1)
    def _finalize():
        cnt = acc_counts[...]
        counts_ref[...] = cnt
        avgp_ref[...] = acc_probs[...] / nt
        ent_ref[...] = acc_ent[...] / nt
        cb = jnp.broadcast_to(cnt, (e, e))
        rr = jax.lax.broadcasted_iota(jnp.int32, (e, e), 0)
        cc = jax.lax.broadcasted_iota(jnp.int32, (e, e), 1)
        ccol = jnp.sum(jnp.where(rr == cc, cb, 0.0), axis=1, keepdims=True)
        less = jnp.sum((cb < ccol).astype(jnp.float32), axis=1, keepdims=True)
        eq = jnp.sum((cb == ccol).astype(jnp.float32), axis=1, keepdims=True)
        num = jnp.sum(ccol * (2.0 * less + eq - e), keepdims=True)
        tot = jnp.sum(cnt, keepdims=True)
        gini_ref[...] = num / (e * tot + 1e-10)
        ctr_ref[...] = acc_keep[...].astype(jnp.int32)
        drop_ref[...] = acc_drop[...]


@jax.jit
def kernel(x, W):
    nt, hidden = x.shape
    e = W.shape[0]
    k = _K
    bt = _BT
    cap = int(nt * k / e * _CAPF)
    grid = nt // bt
    kfn = functools.partial(_router_kernel, bt=bt, e=e, k=k, nt=nt, cap=cap)
    outs = pl.pallas_call(
        kfn,
        grid=(grid,),
        in_specs=[
            pl.BlockSpec((bt, hidden), lambda i: (i, 0)),
            pl.BlockSpec((e, hidden), lambda i: (0, 0)),
        ],
        out_specs=[
            pl.BlockSpec((bt, k), lambda i: (i, 0)),
            pl.BlockSpec((bt, k), lambda i: (i, 0)),
            pl.BlockSpec((bt, k), lambda i: (i, 0)),
            pl.BlockSpec((1, e), lambda i: (0, 0)),
            pl.BlockSpec((1, e), lambda i: (0, 0)),
            pl.BlockSpec((1, 1), lambda i: (0, 0)),
            pl.BlockSpec((1, 1), lambda i: (0, 0)),
            pl.BlockSpec((1, e), lambda i: (0, 0)),
            pl.BlockSpec((1, 1), lambda i: (0, 0)),
        ],
        out_shape=[
            jax.ShapeDtypeStruct((nt, k), jnp.int32),
            jax.ShapeDtypeStruct((nt, k), jnp.float32),
            jax.ShapeDtypeStruct((nt, k), jnp.float32),
            jax.ShapeDtypeStruct((1, e), jnp.float32),
            jax.ShapeDtypeStruct((1, e), jnp.float32),
            jax.ShapeDtypeStruct((1, 1), jnp.float32),
            jax.ShapeDtypeStruct((1, 1), jnp.float32),
            jax.ShapeDtypeStruct((1, e), jnp.int32),
            jax.ShapeDtypeStruct((1, 1), jnp.float32),
        ],
        scratch_shapes=[
            pltpu.VMEM((bt, k), jnp.float32),
            pltpu.VMEM((1, e), jnp.float32),
            pltpu.VMEM((1, e), jnp.float32),
            pltpu.VMEM((1, 1), jnp.float32),
            pltpu.VMEM((1, e), jnp.float32),
            pltpu.VMEM((1, 1), jnp.float32),
        ],
    )(x, W)
    tidx, tw, maskb, counts, avgp, ent, gini, ctr, drop = outs
    return (tidx, tw, maskb,
            counts.reshape(e), avgp.reshape(e),
            ent.reshape(()), gini.reshape(()),
            ctr.reshape(e), drop.reshape(()))


# speedup vs baseline: 2.4883x; 2.4883x over previous
import functools

import jax
import jax.numpy as jnp
from jax.experimental import pallas as pl
from jax.experimental.pallas import tpu as pltpu

_CAPF = 1.25
_K = 8
_BT = 512


def _router_kernel(x_ref, w_ref, tri_ref,
                   idx_ref, wts_ref, mask_ref,
                   counts_ref, avgp_ref, ent_ref, gini_ref, ctr_ref, drop_ref,
                   vbuf_ref, acc_counts, acc_probs, acc_ent, acc_keep,
                   acc_drop,
                   *, bt, e, k, nt, cap):
    i = pl.program_id(0)
    nsteps = pl.num_programs(0)

    @pl.when(i == 0)
    def _init():
        acc_counts[...] = jnp.zeros_like(acc_counts)
        acc_probs[...] = jnp.zeros_like(acc_probs)
        acc_ent[...] = jnp.zeros_like(acc_ent)
        acc_keep[...] = jnp.zeros_like(acc_keep)
        acc_drop[...] = jnp.zeros_like(acc_drop)

    logits = jax.lax.dot_general(w_ref[...], x_ref[...],
                                 (((1,), (1,)), ((), ())),
                                 preferred_element_type=jnp.float32)
    m = jnp.max(logits, axis=0, keepdims=True)
    el = jnp.exp(logits - m)
    probs = el / jnp.sum(el, axis=0, keepdims=True)

    srow = jax.lax.broadcasted_iota(jnp.int32, (e, bt), 0)
    cur = probs
    selected = jnp.zeros((e, bt), jnp.bool_)
    for kk in range(k):
        mk = jnp.max(cur, axis=0, keepdims=True)
        ik = jnp.min(jnp.where(cur == mk, srow, e), axis=0, keepdims=True)
        oh = srow == ik
        idx_ref[kk:kk + 1, :] = ik
        vbuf_ref[kk:kk + 1, :] = mk
        selected = selected | oh
        cur = jnp.where(oh, -jnp.inf, cur)

    h = selected.astype(jnp.float32)
    sv = jnp.sum(jnp.where(selected, probs, 0.0), axis=0, keepdims=True)
    elp = -probs * jnp.log(probs + 1e-10)

    stack = jnp.concatenate([h, probs, elp], axis=0)
    incl = jax.lax.dot_general(stack, tri_ref[...], (((1,), (0,)), ((), ())),
                               preferred_element_type=jnp.float32)
    excl = incl[0:e, :] - h + acc_counts[...]

    keepe = jnp.where(selected & (excl < cap), 1.0, 0.0)
    masksum = jnp.sum(keepe, axis=0, keepdims=True)
    wscale = (1.0 / sv) / (masksum + 1e-10)
    for kk in range(k):
        oh = srow == idx_ref[kk:kk + 1, :]
        mask_k = jnp.sum(jnp.where(oh, keepe, 0.0), axis=0, keepdims=True)
        mask_ref[kk:kk + 1, :] = mask_k
        wts_ref[kk:kk + 1, :] = vbuf_ref[kk:kk + 1, :] * wscale * mask_k

    acc_counts[...] = acc_counts[...] + incl[0:e, bt - 1:bt]
    acc_probs[...] = acc_probs[...] + incl[e:2 * e, bt - 1:bt]
    acc_ent[...] = acc_ent[...] + jnp.sum(incl[2 * e:3 * e, bt - 1:bt],
                                          keepdims=True)
    acc_keep[...] = acc_keep[...] + jnp.sum(keepe, axis=1, keepdims=True)
    acc_drop[...] = acc_drop[...] + (
        float(bt * k) - jnp.sum(masksum, keepdims=True))

    @pl.when(i == nsteps - 1)
    def _finalize():
        cnt = acc_counts[...]
        counts_ref[...] = cnt
        avgp_ref[...] = acc_probs[...] / nt
        ent_ref[...] = acc_ent[...] / nt
        ccol = jnp.broadcast_to(cnt, (e, e))
        rr = jax.lax.broadcasted_iota(jnp.int32, (e, e), 0)
        cc = jax.lax.broadcasted_iota(jnp.int32, (e, e), 1)
        crow = jnp.sum(jnp.where(rr == cc, ccol, 0.0), axis=0, keepdims=True)
        less = jnp.sum((crow < ccol).astype(jnp.float32), axis=1,
                       keepdims=True)
        eq = jnp.sum((crow == ccol).astype(jnp.float32), axis=1,
                     keepdims=True)
        num = jnp.sum(cnt * (2.0 * less + eq - e), keepdims=True)
        tot = jnp.sum(cnt, keepdims=True)
        gini_ref[...] = num / (e * tot + 1e-10)
        ctr_ref[...] = acc_keep[...].astype(jnp.int32)
        drop_ref[...] = acc_drop[...]


@jax.jit
def kernel(x, W):
    nt, hidden = x.shape
    e = W.shape[0]
    k = _K
    bt = _BT
    cap = int(nt * k / e * _CAPF)
    grid = nt // bt
    r = jax.lax.broadcasted_iota(jnp.int32, (bt, bt), 0)
    c = jax.lax.broadcasted_iota(jnp.int32, (bt, bt), 1)
    tri = (r <= c).astype(jnp.float32)
    kfn = functools.partial(_router_kernel, bt=bt, e=e, k=k, nt=nt, cap=cap)
    outs = pl.pallas_call(
        kfn,
        grid=(grid,),
        in_specs=[
            pl.BlockSpec((bt, hidden), lambda i: (i, 0)),
            pl.BlockSpec((e, hidden), lambda i: (0, 0)),
            pl.BlockSpec((bt, bt), lambda i: (0, 0)),
        ],
        out_specs=[
            pl.BlockSpec((k, bt), lambda i: (0, i)),
            pl.BlockSpec((k, bt), lambda i: (0, i)),
            pl.BlockSpec((k, bt), lambda i: (0, i)),
            pl.BlockSpec((e, 1), lambda i: (0, 0)),
            pl.BlockSpec((e, 1), lambda i: (0, 0)),
            pl.BlockSpec((1, 1), lambda i: (0, 0)),
            pl.BlockSpec((1, 1), lambda i: (0, 0)),
            pl.BlockSpec((e, 1), lambda i: (0, 0)),
            pl.BlockSpec((1, 1), lambda i: (0, 0)),
        ],
        out_shape=[
            jax.ShapeDtypeStruct((k, nt), jnp.int32),
            jax.ShapeDtypeStruct((k, nt), jnp.float32),
            jax.ShapeDtypeStruct((k, nt), jnp.float32),
            jax.ShapeDtypeStruct((e, 1), jnp.float32),
            jax.ShapeDtypeStruct((e, 1), jnp.float32),
            jax.ShapeDtypeStruct((1, 1), jnp.float32),
            jax.ShapeDtypeStruct((1, 1), jnp.float32),
            jax.ShapeDtypeStruct((e, 1), jnp.int32),
            jax.ShapeDtypeStruct((1, 1), jnp.float32),
        ],
        scratch_shapes=[
            pltpu.VMEM((k, bt), jnp.float32),
            pltpu.VMEM((e, 1), jnp.float32),
            pltpu.VMEM((e, 1), jnp.float32),
            pltpu.VMEM((1, 1), jnp.float32),
            pltpu.VMEM((e, 1), jnp.float32),
            pltpu.VMEM((1, 1), jnp.float32),
        ],
    )(x, W, tri)
    tidx, tw, maskb, counts, avgp, ent, gini, ctr, drop = outs
    return (tidx.T, tw.T, maskb.T,
            counts.reshape(e), avgp.reshape(e),
            ent.reshape(()), gini.reshape(()),
            ctr.reshape(e), drop.reshape(()))
